# baseline (speedup 1.0000x reference)
import jax
import jax.numpy as jnp
from jax.experimental import pallas as pl
from jax.experimental.pallas import tpu as pltpu
from jax.experimental.pallas import tpu_sc as plsc

BS = 1024
NC = 200
NCU = 50
ED = 128
T = 128

G = 16
SC_WORKERS = 32
BPW = BS // SC_WORKERS


def _tc_body(cs_ref, task_ref, cs3_ref, task3_ref, dyn_ref, mf_ref,
             st_hbm, mfs_hbm, nd_ref, nm_ref, d_ref, dmf_ref, s_ref, mfso_ref,
             st_tmp, mfs_tmp, sem):
    i = pl.program_id(0)
    n = pl.num_programs(0)

    def start_group(grp, par):
        for g in range(G):
            b = grp * G + g
            cs = cs_ref[b]
            cs_al = pl.multiple_of((cs // 8) * 8, 8)
            pltpu.make_async_copy(
                st_hbm.at[b, pl.ds(cs_al, 8), :], st_tmp.at[par, g], sem).start()
            pltpu.make_async_copy(
                mfs_hbm.at[b, pl.ds(cs_al, 8), :], mfs_tmp.at[par, g], sem).start()

    @pl.when(i == 0)
    def _():
        start_group(0, 0)

    @pl.when(i + 1 < n)
    def _():
        start_group(i + 1, (i + 1) % 2)

    lane = jax.lax.broadcasted_iota(jnp.int32, (1, NC), 1)
    cs_col = jnp.reshape(cs3_ref[0], (G, 1))
    task_col = jnp.reshape(task3_ref[0], (G, 1)).astype(jnp.float32)
    onehot2d = (lane == cs_col)
    nd_ref[:, :T, :] = dyn_ref[...]
    nm_ref[:, :T, :] = mf_ref[...]
    nd_ref[:, T, :] = jnp.where(onehot2d, task_col, 0.0)
    nm_ref[:, T, :] = jnp.where(onehot2d, 0.0, -jnp.inf)

    for g in range(G):
        b = i * G + g
        cs = cs_ref[b]
        sel = cs < 128

        def pick(block):
            a = pltpu.roll(block[:, 0:128], -cs, 1)[:, 0:1]
            bb = pltpu.roll(block[:, NC - 128:NC], -(cs - (NC - 128)), 1)[:, 0:1]
            return jnp.where(sel, a, bb)

        d_ref[g, :] = pick(dyn_ref[g])[:, 0]
        dmf_ref[g, :] = pick(mf_ref[g])[:, 0]

    for g in range(G):
        b = i * G + g
        cs = cs_ref[b]
        cs_al = pl.multiple_of((cs // 8) * 8, 8)
        pltpu.make_async_copy(
            st_hbm.at[b, pl.ds(cs_al, 8), :], st_tmp.at[i % 2, g], sem).wait()
        pltpu.make_async_copy(
            mfs_hbm.at[b, pl.ds(cs_al, 8), :], mfs_tmp.at[i % 2, g], sem).wait()

    sub8 = jax.lax.broadcasted_iota(jnp.int32, (8, 1), 0)
    for g in range(G):
        b = i * G + g
        cs = cs_ref[b]
        onehot8 = (sub8 == (cs % 8)).astype(jnp.float32)
        s_ref[g, 0, :] = jnp.sum(st_tmp[i % 2, g] * onehot8, axis=0)
        mfso_ref[g, 0, :] = jnp.sum(mfs_tmp[i % 2, g] * onehot8, axis=0)


def _sc_gather_body(cs_hbm, sth_hbm, sh_out, cs_v, idx_v, r_sh, sem):
    c = jax.lax.axis_index("c")
    s = jax.lax.axis_index("s")
    wid = s * 2 + c
    base = wid * BPW

    pltpu.sync_copy(cs_hbm.at[pl.ds(base, BPW)], cs_v)
    for j in range(BPW // 16):
        off = base + j * 16
        iota = jax.lax.broadcasted_iota(jnp.int32, (16,), 0)
        idx_v[pl.ds(j * 16, 16)] = cs_v[pl.ds(j * 16, 16)] + (iota + off) * NC

    pltpu.async_copy(sth_hbm.at[idx_v], r_sh, sem).wait()
    pltpu.sync_copy(r_sh, sh_out.at[pl.ds(base, BPW)])


def kernel(static, static_h, dynamic, mask_f, mask_fs, couriers_selected,
           sensingtask_selected):
    bs = static.shape[0]
    cs_flat = couriers_selected[:, 0]
    task_flat = sensingtask_selected[:, 0]
    cs3 = cs_flat.reshape(bs // G, 1, G)
    task3 = task_flat.reshape(bs // G, 1, G)

    def at_group(i, cs_r, task_r):
        return (i, 0, 0)

    grid_spec = pltpu.PrefetchScalarGridSpec(
        num_scalar_prefetch=2,
        grid=(bs // G,),
        in_specs=[
            pl.BlockSpec((1, 1, G), at_group),
            pl.BlockSpec((1, 1, G), at_group),
            pl.BlockSpec((G, T, NC), at_group),
            pl.BlockSpec((G, T, NC), at_group),
            pl.BlockSpec(memory_space=pl.ANY),
            pl.BlockSpec(memory_space=pl.ANY),
        ],
        out_specs=[
            pl.BlockSpec((G, T + 1, NC), at_group),
            pl.BlockSpec((G, T + 1, NC), at_group),
            pl.BlockSpec((G, T), lambda i, c, t: (i, 0)),
            pl.BlockSpec((G, T), lambda i, c, t: (i, 0)),
            pl.BlockSpec((G, 1, 2 * NCU), at_group),
            pl.BlockSpec((G, 1, NCU), at_group),
        ],
        scratch_shapes=[
            pltpu.VMEM((2, G, 8, 2 * NCU), jnp.float32),
            pltpu.VMEM((2, G, 8, NCU), jnp.float32),
            pltpu.SemaphoreType.DMA,
        ],
    )

    nd, nm, d, mf, s, mfs = pl.pallas_call(
        _tc_body,
        grid_spec=grid_spec,
        out_shape=[
            jax.ShapeDtypeStruct((bs, T + 1, NC), jnp.float32),
            jax.ShapeDtypeStruct((bs, T + 1, NC), jnp.float32),
            jax.ShapeDtypeStruct((bs, T), jnp.float32),
            jax.ShapeDtypeStruct((bs, T), jnp.float32),
            jax.ShapeDtypeStruct((bs, 1, 2 * NCU), jnp.float32),
            jax.ShapeDtypeStruct((bs, 1, NCU), jnp.float32),
        ],
    )(cs_flat, task_flat, cs3, task3, dynamic, mask_f, static, mask_fs)

    sc_call = pl.kernel(
        _sc_gather_body,
        out_type=[
            jax.ShapeDtypeStruct((bs, ED), jnp.float32),
        ],
        mesh=plsc.VectorSubcoreMesh(core_axis_name="c", subcore_axis_name="s",
                                    num_cores=2, num_subcores=16),
        scratch_types=[
            pltpu.VMEM((BPW,), jnp.int32),
            pltpu.VMEM((BPW,), jnp.int32),
            pltpu.VMEM((BPW, ED), jnp.float32),
            pltpu.SemaphoreType.DMA,
        ],
    )
    (sh_f,) = sc_call(cs_flat, static_h.reshape(bs * NC, ED))

    return (s, sh_f[:, None, :], d[:, :, None], mf[:, :, None], mfs, nd, nm)


# speedup vs baseline: 1.1329x; 1.1329x over previous
import jax
import jax.numpy as jnp
from jax.experimental import pallas as pl
from jax.experimental.pallas import tpu as pltpu
from jax.experimental.pallas import tpu_sc as plsc

BS = 1024
NC = 200
NCU = 50
ED = 128
T = 128

G = 16
SC_WORKERS = 32
BPW = BS // SC_WORKERS


def _tc_body(cs_ref, task_ref, dyn_ref, mf_ref, nd_ref, nm_ref, d_ref, dmf_ref):
    i = pl.program_id(0)

    nd_ref[:, :T, :] = dyn_ref[...]
    nm_ref[:, :T, :] = mf_ref[...]

    lane = jax.lax.broadcasted_iota(jnp.int32, (1, NC), 1)
    sub = jax.lax.broadcasted_iota(jnp.int32, (NC, 1), 0)

    for g in range(G):
        b = i * G + g
        cs = cs_ref[b]
        task = task_ref[b].astype(jnp.float32)

        onehot_row = (lane == cs)
        nd_ref[g, T:T + 1, :] = jnp.where(onehot_row, task, 0.0)
        nm_ref[g, T:T + 1, :] = jnp.where(onehot_row, 0.0, -jnp.inf)

        onehot_col = (sub == cs).astype(jnp.float32)
        d_ref[g, :, :] = jax.lax.dot(
            dyn_ref[g], onehot_col,
            precision=jax.lax.Precision.HIGHEST,
            preferred_element_type=jnp.float32)
        dmf_ref[g, :, :] = jax.lax.dot(
            mf_ref[g], onehot_col,
            precision=jax.lax.Precision.HIGHEST,
            preferred_element_type=jnp.float32)


def _sc_gather_body(cs_hbm, st_hbm, sth_hbm, mfs_hbm, s_out, sh_out, mfs_out,
                    cs_v, idx_v, r_s, r_sh, r_mfs, sem, sem2):
    c = jax.lax.axis_index("c")
    s = jax.lax.axis_index("s")
    wid = s * 2 + c
    base = wid * BPW

    pltpu.sync_copy(cs_hbm.at[pl.ds(base, BPW)], cs_v)
    for j in range(BPW // 16):
        off = base + j * 16
        iota = jax.lax.broadcasted_iota(jnp.int32, (16,), 0)
        idx_v[pl.ds(j * 16, 16)] = cs_v[pl.ds(j * 16, 16)] + (iota + off) * NC

    sh_dma = pltpu.async_copy(sth_hbm.at[idx_v], r_sh, sem)

    fired = []
    for j16 in range(BPW // 16):
        vec = idx_v[pl.ds(j16 * 16, 16)]
        for l in range(16):
            j = j16 * 16 + l
            row = vec[l]
            fired.append(pltpu.async_copy(
                st_hbm.at[pl.ds(row, 1)], r_s.at[pl.ds(j, 1)], sem2))
            fired.append(pltpu.async_copy(
                mfs_hbm.at[pl.ds(row, 1)], r_mfs.at[pl.ds(j, 1)], sem2))
    sh_dma.wait()
    for dma in fired:
        dma.wait()

    pltpu.sync_copy(r_s, s_out.at[pl.ds(base, BPW)])
    pltpu.sync_copy(r_sh, sh_out.at[pl.ds(base, BPW)])
    pltpu.sync_copy(r_mfs, mfs_out.at[pl.ds(base, BPW)])


def kernel(static, static_h, dynamic, mask_f, mask_fs, couriers_selected,
           sensingtask_selected):
    bs = static.shape[0]
    cs_flat = couriers_selected[:, 0]
    task_flat = sensingtask_selected[:, 0]

    def at_b(i, cs_r, task_r):
        return (i, 0, 0)

    grid_spec = pltpu.PrefetchScalarGridSpec(
        num_scalar_prefetch=2,
        grid=(bs // G,),
        in_specs=[
            pl.BlockSpec((G, T, NC), at_b),
            pl.BlockSpec((G, T, NC), at_b),
        ],
        out_specs=[
            pl.BlockSpec((G, T + 1, NC), at_b),
            pl.BlockSpec((G, T + 1, NC), at_b),
            pl.BlockSpec((G, T, 1), at_b),
            pl.BlockSpec((G, T, 1), at_b),
        ],
    )

    nd, nm, d, mf = pl.pallas_call(
        _tc_body,
        grid_spec=grid_spec,
        out_shape=[
            jax.ShapeDtypeStruct((bs, T + 1, NC), jnp.float32),
            jax.ShapeDtypeStruct((bs, T + 1, NC), jnp.float32),
            jax.ShapeDtypeStruct((bs, T, 1), jnp.float32),
            jax.ShapeDtypeStruct((bs, T, 1), jnp.float32),
        ],
    )(cs_flat, task_flat, dynamic, mask_f)

    sc_call = pl.kernel(
        _sc_gather_body,
        out_type=[
            jax.ShapeDtypeStruct((bs, 2 * NCU), jnp.float32),
            jax.ShapeDtypeStruct((bs, ED), jnp.float32),
            jax.ShapeDtypeStruct((bs, NCU), jnp.float32),
        ],
        mesh=plsc.VectorSubcoreMesh(core_axis_name="c", subcore_axis_name="s",
                                    num_cores=2, num_subcores=16),
        scratch_types=[
            pltpu.VMEM((BPW,), jnp.int32),
            pltpu.VMEM((BPW,), jnp.int32),
            pltpu.VMEM((BPW, 2 * NCU), jnp.float32),
            pltpu.VMEM((BPW, ED), jnp.float32),
            pltpu.VMEM((BPW, NCU), jnp.float32),
            pltpu.SemaphoreType.DMA,
            pltpu.SemaphoreType.DMA,
        ],
    )
    s_f, sh_f, mfs_f = sc_call(
        cs_flat,
        static.reshape(bs * NC, 2 * NCU),
        static_h.reshape(bs * NC, ED),
        mask_fs.reshape(bs * NC, NCU),
    )

    return (s_f[:, None, :], sh_f[:, None, :], d, mf, mfs_f[:, None, :], nd, nm)
